# Initial kernel scaffold; baseline (speedup 1.0000x reference)
import jax
import jax.numpy as jnp
from jax.experimental import pallas as pl


def kernel(X, z, dt_val, graph_logits, alpha, beta, ws1, bs1, ws2, bs2, w1r, b1r, w2r, b2r):
    raise NotImplementedError("write your pallas kernel here")


# speedup vs baseline: 11.7635x; 11.7635x over previous
import jax
import jax.numpy as jnp
from jax.experimental import pallas as pl
from jax.experimental.pallas import tpu as pltpu

_M, _N = 8, 4
_SH, _RH = 1024, 256
_TB = 1024


def _prep_kernel(g_ref, gT_ref, a_ref, b_ref, c_ref):
    g = g_ref[...]
    gT = gT_ref[...]
    s = 0.5 * (g + gT)
    w = jax.nn.sigmoid(s)
    ii = jax.lax.broadcasted_iota(jnp.int32, (_M, _N, _N), 1)
    jj = jax.lax.broadcasted_iota(jnp.int32, (_M, _N, _N), 2)
    eye = (ii == jj)
    w = jnp.where(eye, 0.0, w)
    deg = jnp.sum(w, axis=2, keepdims=True)
    alpha = a_ref[...]
    beta = b_ref[...]
    c_ref[...] = alpha * jnp.where(eye, deg, 0.0) + beta * w


def _main_kernel(dt_ref, xt_ref, z_ref, ct_ref, ws1_ref, bs1_ref, ws2_ref,
                 bs2_ref, w1rT_ref, b1rT_ref, w2r_ref, b2r_ref, out_ref):
    tb = xt_ref.shape[1]
    z = z_ref[0:1, :]
    modes = jax.lax.broadcasted_iota(jnp.int32, (_M, tb), 0)
    oh = (modes == z).astype(jnp.float32)

    w1z = jnp.dot(w1rT_ref[...], oh, preferred_element_type=jnp.float32)
    b1z = jnp.dot(b1rT_ref[...], oh, preferred_element_type=jnp.float32)
    cz = jnp.dot(ct_ref[...], oh, preferred_element_type=jnp.float32)
    b2z = jnp.dot(b2r_ref[...], oh, preferred_element_type=jnp.float32)

    dt = dt_ref[0:1, 0:1]
    ws1 = ws1_ref[...]
    bs1 = bs1_ref[...]
    ws2 = ws2_ref[...]
    bs2 = bs2_ref[...]
    w2r = w2r_ref[...]

    for n in range(_N):
        x = xt_ref[n:n + 1, :]
        t = jnp.tanh(ws1 * x + bs1)
        shared = jnp.dot(ws2, t, preferred_element_type=jnp.float32) + bs2
        t2 = jnp.tanh(w1z * x + b1z)
        rall = jnp.dot(w2r, t2, preferred_element_type=jnp.float32)
        res = jnp.sum(rall * oh, axis=0, keepdims=True) + b2z
        cpl = cz[_N * n:_N * n + 1, :] * xt_ref[0:1, :]
        for j in range(1, _N):
            cpl = cpl + cz[_N * n + j:_N * n + j + 1, :] * xt_ref[j:j + 1, :]
        out_ref[n:n + 1, :] = x + dt * (shared + res + cpl)


def kernel(X, z, dt_val, graph_logits, alpha, beta, ws1, bs1, ws2, bs2,
           w1r, b1r, w2r, b2r):
    B = X.shape[0]
    f32 = jnp.float32

    gT = jnp.swapaxes(graph_logits, 1, 2)
    cmat = pl.pallas_call(
        _prep_kernel,
        out_shape=jax.ShapeDtypeStruct((_M, _N, _N), f32),
    )(graph_logits, gT, alpha.reshape(_M, 1, 1), beta.reshape(_M, 1, 1))
    ct = cmat.reshape(_M, _N * _N).T

    xt = X.T
    z2 = z.astype(jnp.int32).reshape(1, B)
    dt = jnp.asarray(dt_val, f32).reshape(1, 1)
    ws1c = ws1.reshape(_SH, 1)
    bs1c = bs1.reshape(_SH, 1)
    ws2r = ws2.reshape(1, _SH)
    bs2c = bs2.reshape(1, 1)
    w1rT = w1r.T
    b1rT = b1r.T
    b2rr = b2r.reshape(1, _M)

    grid = (B // _TB,)
    full = lambda shape: pl.BlockSpec(shape, lambda i: (0,) * len(shape))
    outT = pl.pallas_call(
        _main_kernel,
        grid=grid,
        in_specs=[
            full((1, 1)),
            pl.BlockSpec((_N, _TB), lambda i: (0, i)),
            pl.BlockSpec((1, _TB), lambda i: (0, i)),
            full((_N * _N, _M)),
            full((_SH, 1)), full((_SH, 1)),
            full((1, _SH)), full((1, 1)),
            full((_RH, _M)), full((_RH, _M)),
            full((_M, _RH)), full((1, _M)),
        ],
        out_specs=pl.BlockSpec((_N, _TB), lambda i: (0, i)),
        out_shape=jax.ShapeDtypeStruct((_N, B), f32),
        compiler_params=pltpu.CompilerParams(
            dimension_semantics=("arbitrary",),
        ),
    )(dt, xt, z2, ct, ws1c, bs1c, ws2r, bs2c, w1rT, b1rT, w2r, b2rr)
    return outT.T
